# baseline (speedup 1.0000x reference)
import math

import jax
import jax.numpy as jnp
from jax.experimental import pallas as pl

S = 30.0
M = 0.5
COS_M = math.cos(M)
SIN_M = math.sin(M)
TH = math.cos(math.pi - M)
MM = math.sin(math.pi - M) * M

BC = 2048


def _arcface_body(lab_ref, cos_ref, out_ref):
    j = pl.program_id(0)
    cos = cos_ref[...]
    lab = lab_ref[...]
    col = jax.lax.broadcasted_iota(jnp.int32, cos.shape, 1) + j * BC
    sine = jnp.sqrt(jnp.maximum(1.0 - cos * cos, 0.0))
    phi = cos * COS_M - sine * SIN_M
    phi = jnp.where(cos > TH, phi, cos - MM)
    out_ref[...] = jnp.where(col == lab, phi, cos) * S


def kernel(cosine, label):
    B, C = cosine.shape
    lab2 = label.astype(jnp.int32).reshape(B, 1)
    return pl.pallas_call(
        _arcface_body,
        grid=(pl.cdiv(C, BC),),
        in_specs=[
            pl.BlockSpec((B, 1), lambda j: (0, 0)),
            pl.BlockSpec((B, BC), lambda j: (0, j)),
        ],
        out_specs=pl.BlockSpec((B, BC), lambda j: (0, j)),
        out_shape=jax.ShapeDtypeStruct((B, C), cosine.dtype),
    )(lab2, cosine)


# speedup vs baseline: 1.1594x; 1.1594x over previous
import math

import jax
import jax.numpy as jnp
from jax.experimental import pallas as pl

S = 30.0
M = 0.5
COS_M = math.cos(M)
SIN_M = math.sin(M)
TH = math.cos(math.pi - M)
MM = math.sin(math.pi - M) * M

BC = 2048


def _arcface_body(lab_ref, cos_ref, out_ref):
    j = pl.program_id(0)
    cos = cos_ref[...]
    lab_local = lab_ref[...] - j * BC
    col = jax.lax.broadcasted_iota(jnp.int32, cos.shape, 1)
    mask = col == lab_local
    sc = cos * S
    t = jnp.sum(jnp.where(mask, cos, 0.0), axis=1, keepdims=True)
    sine = jnp.sqrt(jnp.maximum(1.0 - t * t, 0.0))
    phi_s = t * (S * COS_M) - sine * (S * SIN_M)
    phi_s = jnp.where(t > TH, phi_s, t * S - S * MM)
    out_ref[...] = jnp.where(mask, phi_s, sc)


def kernel(cosine, label):
    B, C = cosine.shape
    lab2 = label.astype(jnp.int32).reshape(B, 1)
    return pl.pallas_call(
        _arcface_body,
        grid=(pl.cdiv(C, BC),),
        in_specs=[
            pl.BlockSpec((B, 1), lambda j: (0, 0)),
            pl.BlockSpec((B, BC), lambda j: (0, j)),
        ],
        out_specs=pl.BlockSpec((B, BC), lambda j: (0, j)),
        out_shape=jax.ShapeDtypeStruct((B, C), cosine.dtype),
    )(lab2, cosine)
